# Initial kernel scaffold; baseline (speedup 1.0000x reference)
import jax
import jax.numpy as jnp
from jax.experimental import pallas as pl


def kernel(x, pe):
    raise NotImplementedError("write your pallas kernel here")


# speedup vs baseline: 1.6831x; 1.6831x over previous
import jax
import jax.numpy as jnp
from jax.experimental import pallas as pl
from jax.experimental.pallas import tpu as pltpu


def _add_body(x_ref, pe_ref, o_ref):
    o_ref[...] = x_ref[...] + pe_ref[...]


def kernel(x, pe):
    B, L, D = x.shape
    ROWS = 512
    nL = L // ROWS
    xf = x.reshape(B * L, D)
    out = pl.pallas_call(
        _add_body,
        grid=(nL, B),
        in_specs=[
            pl.BlockSpec((ROWS, D), lambda j, b: (b * nL + j, 0)),
            pl.BlockSpec((ROWS, D), lambda j, b: (j, 0)),
        ],
        out_specs=pl.BlockSpec((ROWS, D), lambda j, b: (b * nL + j, 0)),
        out_shape=jax.ShapeDtypeStruct((B * L, D), x.dtype),
    )(xf, pe)
    return out.reshape(B, L, D)
